# Initial kernel scaffold; baseline (speedup 1.0000x reference)
import jax
import jax.numpy as jnp
from jax.experimental import pallas as pl


def kernel(x, table, W, b):
    raise NotImplementedError("write your pallas kernel here")


# speedup vs baseline: 6.3479x; 6.3479x over previous
import functools

import jax
import jax.numpy as jnp
from jax import lax
from jax.experimental import pallas as pl
from jax.experimental.pallas import tpu as pltpu
from jax.experimental.pallas import tpu_sc as plsc

NUM_CORES = 2
NUM_SUBCORES = 16
NUM_WORKERS = NUM_CORES * NUM_SUBCORES

STREAM = 128
STREAMS_PER_SUPER = 10
SUPER = STREAM * STREAMS_PER_SUPER


def _gather_body(per_w, n_super, d, table_hbm, idx_hbm, out_hbm,
                 idx_v, rows0, rows1, sem0, sem1):
    wid = lax.axis_index("s") * NUM_CORES + lax.axis_index("c")
    base = wid * per_w
    pltpu.sync_copy(idx_hbm.at[wid], idx_v)

    bufs = (rows0, rows1)
    sems = (sem0, sem1)

    def issue(sup):
        buf = bufs[sup % 2]
        sem = sems[sup % 2]
        cps = []
        for j in range(STREAMS_PER_SUPER):
            s = sup * STREAMS_PER_SUPER + j
            cps.append(pltpu.async_copy(
                table_hbm.at[idx_v.at[s]],
                buf.at[pl.ds(j * STREAM, STREAM)],
                sem))
        return cps

    pending = [issue(0), None]
    for sup in range(n_super):
        nxt = sup + 1
        if nxt < n_super:
            pending[nxt % 2] = issue(nxt)
        for cp in pending[sup % 2]:
            cp.wait()
        pltpu.sync_copy(bufs[sup % 2],
                        out_hbm.at[pl.ds(base + sup * SUPER, SUPER)])


def _sc_gather(table, idx):
    n = idx.shape[0]
    d = table.shape[1]
    per_w = n // NUM_WORKERS
    n_super = per_w // SUPER
    assert per_w % SUPER == 0
    mesh = plsc.VectorSubcoreMesh(core_axis_name="c", subcore_axis_name="s")
    f = pl.kernel(
        functools.partial(_gather_body, per_w, n_super, d),
        out_type=jax.ShapeDtypeStruct((n, d), jnp.float32),
        mesh=mesh,
        scratch_types=[
            pltpu.VMEM((per_w // STREAM, STREAM), jnp.int32),
            pltpu.VMEM((SUPER, d), jnp.float32),
            pltpu.VMEM((SUPER, d), jnp.float32),
            pltpu.SemaphoreType.DMA,
            pltpu.SemaphoreType.DMA,
        ],
        compiler_params=pltpu.CompilerParams(use_tc_tiling_on_sc=False),
    )
    return f(table, idx.reshape(NUM_WORKERS, per_w // STREAM, STREAM))


def _matmul_body(emb_ref, w_ref, b_ref, out_ref):
    out_ref[...] = jnp.dot(
        emb_ref[...], w_ref[...], preferred_element_type=jnp.float32
    ) + b_ref[...]


def _tc_matmul(emb, W, b):
    n, d = emb.shape
    o = W.shape[1]
    blk = 4096
    return pl.pallas_call(
        _matmul_body,
        grid=(n // blk,),
        in_specs=[
            pl.BlockSpec((blk, d), lambda i: (i, 0)),
            pl.BlockSpec((d, o), lambda i: (0, 0)),
            pl.BlockSpec((1, o), lambda i: (0, 0)),
        ],
        out_specs=pl.BlockSpec((blk, o), lambda i: (i, 0)),
        out_shape=jax.ShapeDtypeStruct((n, o), jnp.float32),
    )(emb, W, b.reshape(1, o))


def kernel(x, table, W, b):
    bsz, seq = x.shape
    xf = x.reshape(-1).astype(jnp.int32)
    emb = _sc_gather(table, xf)
    out = _tc_matmul(emb, W, b)
    return out.reshape(bsz, seq, W.shape[1])
